# baseline (speedup 1.0000x reference)
import functools

import jax
import jax.numpy as jnp
import numpy as np
from jax import lax
from jax.experimental import pallas as pl
from jax.experimental.pallas import tpu as pltpu
from jax.experimental.pallas import tpu_sc as plsc

_K = 16
_NUM_HEADS = 16
_HEADS_PER_BLOCK = 2
_ROW_BLOCK = 64


def _dist_body(x_ref, dist_ref):
    xb = x_ref[0]
    sq = jnp.sum(xb * xb, axis=0)
    xbb = xb.astype(jnp.bfloat16)
    prod = lax.dot_general(xbb, xbb, (((0,), (0,)), ((), ())),
                           preferred_element_type=jnp.float32)
    dist_ref[...] = sq[:, None] - 2.0 * prod + sq[None, :]


def _pairwise_dist(x):
    b, c, n = x.shape
    return pl.pallas_call(
        _dist_body,
        grid=(b,),
        in_specs=[pl.BlockSpec((1, c, n), lambda i: (i, 0, 0))],
        out_specs=pl.BlockSpec((n, n), lambda i: (i, 0)),
        out_shape=jax.ShapeDtypeStruct((b * n, n), jnp.float32),
    )(x)


def _topk_sc(dist_flat):
    r, n = dist_flat.shape
    k = _K
    info = plsc.get_sparse_core_info()
    nw = info.num_cores * info.num_subcores
    rows_per_w = r // nw
    rb = _ROW_BLOCK
    nblocks = rows_per_w // rb
    nchunks = n // k
    nlad = 8
    mesh = plsc.VectorSubcoreMesh(core_axis_name="c", subcore_axis_name="s")

    @functools.partial(
        pl.kernel,
        mesh=mesh,
        out_type=jax.ShapeDtypeStruct((r, k), jnp.int32),
        scratch_types=[
            pltpu.VMEM((n,), jnp.float32),
            pltpu.VMEM((k,), jnp.int32),
        ],
    )
    def tk(dist_hbm, out_hbm, row_v, idx_v):
        wid = lax.axis_index("s") * info.num_cores + lax.axis_index("c")
        base = wid * rows_per_w
        lane = lax.iota(jnp.int32, k)
        inf = jnp.float32(np.inf)

        def bfly_min(v):
            for st in (1, 2, 4, 8):
                v = jnp.minimum(v, jnp.take(v, lane ^ st))
            return v

        def row_body(i, carry):
            rr = base + i
            pltpu.sync_copy(dist_hbm.at[rr], row_v)
            st0 = tuple(row_v[pl.ds(j * k, k)] for j in range(nlad)) + \
                  tuple(lane + j * k for j in range(nlad))

            def chunk_body(j, st):
                tv = list(st[:nlad])
                iv = list(st[nlad:])
                c = row_v[pl.ds(j * k, k)]
                ci = lane + j * k
                for q in range(nlad):
                    m = c < tv[q]
                    ntv = jnp.where(m, c, tv[q])
                    niv = jnp.where(m, ci, iv[q])
                    c = jnp.where(m, tv[q], c)
                    ci = jnp.where(m, iv[q], ci)
                    tv[q], iv[q] = ntv, niv
                return tuple(tv) + tuple(iv)

            st = lax.fori_loop(nlad, nchunks, chunk_body, st0)
            tv = list(st[:nlad])
            iv = list(st[nlad:])
            one = jnp.ones_like(lane)
            zero = jnp.zeros_like(lane)
            big = jnp.float32(1e30)
            res = lane
            for r in range(k):
                m8 = tv[0]
                for q in range(1, nlad):
                    m8 = jnp.minimum(m8, tv[q])
                g = bfly_min(m8)
                eqs = [jnp.where(tv[q] == g, one, zero) for q in range(nlad)]
                anyeq = eqs[0]
                for q in range(1, nlad):
                    anyeq = jnp.maximum(anyeq, eqs[q])
                fl = anyeq * lane + (one - anyeq) * k
                for stp in (1, 2, 4, 8):
                    fl = jnp.minimum(fl, jnp.take(fl, lane ^ stp))
                islane = jnp.where(lane == fl, one, zero)
                taken = zero
                winner = zero
                for q in range(nlad):
                    hit = eqs[q] * islane * (one - taken)
                    winner = winner + hit * iv[q]
                    tv[q] = tv[q] + hit.astype(jnp.float32) * big
                    taken = taken + hit
                wsplat = jnp.take(winner, fl)
                res = jnp.where(lane == r, wsplat, res)
            idx_v[...] = res
            pltpu.sync_copy(idx_v, out_hbm.at[rr])
            return carry

        lax.fori_loop(0, rows_per_w, row_body, 0)

    return tk(dist_flat)


def _gather_body(x_ref, idx_ref, out_ref):
    xb = x_ref[0]
    idxb = idx_ref[0]
    n = xb.shape[1]
    col = lax.broadcasted_iota(jnp.int32, (n, n), 1)
    acc = jnp.zeros((n, n), jnp.float32)
    for t in range(_K):
        acc = acc + (col == idxb[:, t:t + 1]).astype(jnp.float32)
    out_ref[0] = lax.dot_general(xb, acc, (((1,), (1,)), ((), ())),
                                 preferred_element_type=jnp.float32,
                                 precision=lax.Precision.HIGHEST) * (1.0 / _K)


def _gather_mean_t(x, idx):
    b, c, n = x.shape
    return pl.pallas_call(
        _gather_body,
        grid=(b,),
        in_specs=[
            pl.BlockSpec((1, c, n), lambda i: (i, 0, 0)),
            pl.BlockSpec((1, n, _K), lambda i: (i, 0, 0)),
        ],
        out_specs=pl.BlockSpec((1, c, n), lambda i: (i, 0, 0)),
        out_shape=jax.ShapeDtypeStruct((b, c, n), jnp.float32),
    )(x, idx)


def _mha_body(xq_ref, xe_ref, wq_ref, wk_ref, wv_ref, wo_ref,
              bq_ref, bk_ref, bv_ref, bo_ref, out_ref, *, dh):
    hp = pl.program_id(1)
    l = xq_ref.shape[1]
    dn = (((1,), (1,)), ((), ()))
    xq = xq_ref[0]
    xqb = xq.astype(jnp.bfloat16)
    xe = xe_ref[0]
    q2 = lax.dot_general(xqb, wq_ref[...], dn,
                         preferred_element_type=jnp.float32) + bq_ref[0]
    k2 = lax.dot_general(xe, wk_ref[...], dn,
                         preferred_element_type=jnp.float32) + bk_ref[0]
    v2 = lax.dot_general(xe, wv_ref[...], dn,
                         preferred_element_type=jnp.float32) + bv_ref[0]
    scale = 1.0 / np.sqrt(dh)
    outs = []
    for h in range(_HEADS_PER_BLOCK):
        sl = slice(h * dh, (h + 1) * dh)
        qh = (q2[:, sl] * scale).astype(jnp.bfloat16)
        kh = k2[:, sl].astype(jnp.bfloat16)
        s = lax.dot_general(qh, kh, dn, preferred_element_type=jnp.float32)
        m = jnp.max(s, axis=1, keepdims=True)
        p = jnp.exp(s - m)
        a = (p / jnp.sum(p, axis=1, keepdims=True)).astype(jnp.bfloat16)
        vh = v2[:, sl].astype(jnp.bfloat16)
        outs.append(lax.dot_general(a, vh, (((1,), (0,)), ((), ())),
                                    preferred_element_type=jnp.float32))
    o2 = jnp.concatenate(outs, axis=1).astype(jnp.bfloat16)
    proj = lax.dot_general(o2, wo_ref[...], dn,
                           preferred_element_type=jnp.float32)

    @pl.when(hp == 0)
    def _():
        out_ref[0, :l, :] = xq
        out_ref[0, l:, :] = proj + bo_ref[0][None, :]

    @pl.when(hp != 0)
    def _():
        out_ref[0, l:, :] += proj


def kernel(x, x_enc, in_proj_weight, in_proj_bias, out_proj_weight, out_proj_bias):
    b, c, n = x.shape
    s, e = x_enc.shape[1], x_enc.shape[2]
    l = c
    dh = e // _NUM_HEADS
    hb = _HEADS_PER_BLOCK
    w = hb * dh
    nhp = _NUM_HEADS // hb

    xe = x_enc.astype(jnp.bfloat16)
    wq = in_proj_weight[:e].astype(jnp.bfloat16)
    wk = in_proj_weight[e:2 * e].astype(jnp.bfloat16)
    wv = in_proj_weight[2 * e:].astype(jnp.bfloat16)
    wo = out_proj_weight.astype(jnp.bfloat16)
    bq = in_proj_bias[:e].reshape(nhp, 1, w)
    bk = in_proj_bias[e:2 * e].reshape(nhp, 1, w)
    bv = in_proj_bias[2 * e:].reshape(nhp, 1, w)
    bo = out_proj_bias.reshape(1, e)

    grid = (b, nhp)

    dist = _pairwise_dist(x)
    idx = _topk_sc(dist)
    xknn_t = _gather_mean_t(x, idx.reshape(b, n, _K))
    xq = jnp.stack([x, xknn_t], axis=3).reshape(b, c, 2 * n)

    out = pl.pallas_call(
        functools.partial(_mha_body, dh=dh),
        grid=grid,
        in_specs=[
            pl.BlockSpec((1, l, e), lambda i, j: (i, 0, 0)),
            pl.BlockSpec((1, s, e), lambda i, j: (i, 0, 0)),
            pl.BlockSpec((w, e), lambda i, j: (j, 0)),
            pl.BlockSpec((w, e), lambda i, j: (j, 0)),
            pl.BlockSpec((w, e), lambda i, j: (j, 0)),
            pl.BlockSpec((e, w), lambda i, j: (0, j)),
            pl.BlockSpec((1, 1, w), lambda i, j: (j, 0, 0)),
            pl.BlockSpec((1, 1, w), lambda i, j: (j, 0, 0)),
            pl.BlockSpec((1, 1, w), lambda i, j: (j, 0, 0)),
            pl.BlockSpec((1, e), lambda i, j: (0, 0)),
        ],
        out_specs=pl.BlockSpec((1, 2 * l, e), lambda i, j: (i, 0, 0)),
        out_shape=jax.ShapeDtypeStruct((b, 2 * l, e), jnp.float32),
        compiler_params=pltpu.CompilerParams(
            dimension_semantics=("parallel", "arbitrary"),
            vmem_limit_bytes=50 * 1024 * 1024,
        ),
    )(xq, xe, wq, wk, wv, wo, bq, bk, bv, bo)
    return out


# speedup vs baseline: 1.0005x; 1.0005x over previous
import functools

import jax
import jax.numpy as jnp
import numpy as np
from jax import lax
from jax.experimental import pallas as pl
from jax.experimental.pallas import tpu as pltpu
from jax.experimental.pallas import tpu_sc as plsc

_K = 16
_NUM_HEADS = 16
_HEADS_PER_BLOCK = 2


def _dist_body(x_ref, dist_ref):
    xb = x_ref[0]
    sq = jnp.sum(xb * xb, axis=0)
    xbb = xb.astype(jnp.bfloat16)
    prod = lax.dot_general(xbb, xbb, (((0,), (0,)), ((), ())),
                           preferred_element_type=jnp.float32)
    dist_ref[...] = sq[:, None] - 2.0 * prod + sq[None, :]


def _pairwise_dist(x):
    b, c, n = x.shape
    return pl.pallas_call(
        _dist_body,
        grid=(b,),
        in_specs=[pl.BlockSpec((1, c, n), lambda i: (i, 0, 0))],
        out_specs=pl.BlockSpec((n, n), lambda i: (i, 0)),
        out_shape=jax.ShapeDtypeStruct((b * n, n), jnp.float32),
    )(x)


def _topk_sc(dist_flat):
    r, n = dist_flat.shape
    k = _K
    info = plsc.get_sparse_core_info()
    nw = info.num_cores * info.num_subcores
    rows_per_w = r // nw
    nchunks = n // k
    nlad = 8
    mesh = plsc.VectorSubcoreMesh(core_axis_name="c", subcore_axis_name="s")

    @functools.partial(
        pl.kernel,
        mesh=mesh,
        out_type=jax.ShapeDtypeStruct((r, k), jnp.int32),
        scratch_types=[
            pltpu.VMEM((n,), jnp.float32),
            pltpu.VMEM((k,), jnp.int32),
        ],
    )
    def tk(dist_hbm, out_hbm, row_v, idx_v):
        wid = lax.axis_index("s") * info.num_cores + lax.axis_index("c")
        base = wid * rows_per_w
        lane = lax.iota(jnp.int32, k)

        def bfly_min(v):
            for st in (1, 2, 4, 8):
                v = jnp.minimum(v, jnp.take(v, lane ^ st))
            return v

        def row_body(i, carry):
            rr = base + i
            pltpu.sync_copy(dist_hbm.at[rr], row_v)
            st0 = tuple(row_v[pl.ds(j * k, k)] for j in range(nlad)) + \
                  tuple(lane + j * k for j in range(nlad))

            def chunk_body(j, st):
                tv = list(st[:nlad])
                iv = list(st[nlad:])
                c = row_v[pl.ds(j * k, k)]
                ci = lane + j * k
                for q in range(nlad):
                    m = c < tv[q]
                    ntv = jnp.where(m, c, tv[q])
                    niv = jnp.where(m, ci, iv[q])
                    c = jnp.where(m, tv[q], c)
                    ci = jnp.where(m, iv[q], ci)
                    tv[q], iv[q] = ntv, niv
                return tuple(tv) + tuple(iv)

            st = lax.fori_loop(nlad, nchunks, chunk_body, st0)
            tv = list(st[:nlad])
            iv = list(st[nlad:])
            one = jnp.ones_like(lane)
            zero = jnp.zeros_like(lane)
            big = jnp.float32(1e30)
            res = lane
            for r in range(k):
                m8 = tv[0]
                for q in range(1, nlad):
                    m8 = jnp.minimum(m8, tv[q])
                g = bfly_min(m8)
                eqs = [jnp.where(tv[q] == g, one, zero) for q in range(nlad)]
                anyeq = eqs[0]
                for q in range(1, nlad):
                    anyeq = jnp.maximum(anyeq, eqs[q])
                fl = anyeq * lane + (one - anyeq) * k
                for stp in (1, 2, 4, 8):
                    fl = jnp.minimum(fl, jnp.take(fl, lane ^ stp))
                islane = jnp.where(lane == fl, one, zero)
                taken = zero
                winner = zero
                for q in range(nlad):
                    hit = eqs[q] * islane * (one - taken)
                    winner = winner + hit * iv[q]
                    tv[q] = tv[q] + hit.astype(jnp.float32) * big
                    taken = taken + hit
                wsplat = jnp.take(winner, fl)
                res = jnp.where(lane == r, wsplat, res)
            idx_v[...] = res
            pltpu.sync_copy(idx_v, out_hbm.at[rr])
            return carry

        lax.fori_loop(0, rows_per_w, row_body, 0)

    return tk(dist_flat)


def _gather_body(x_ref, idx_ref, out_ref):
    xb = x_ref[0]
    idxb = idx_ref[0]
    n = xb.shape[1]
    col = lax.broadcasted_iota(jnp.int32, (n, n), 1)
    acc = jnp.zeros((n, n), jnp.float32)
    for t in range(_K):
        acc = acc + (col == idxb[:, t:t + 1]).astype(jnp.float32)
    out_ref[0] = lax.dot_general(xb, acc, (((1,), (1,)), ((), ())),
                                 preferred_element_type=jnp.float32,
                                 precision=lax.Precision.HIGHEST) * (1.0 / _K)


def _gather_mean_t(x, idx):
    b, c, n = x.shape
    return pl.pallas_call(
        _gather_body,
        grid=(b,),
        in_specs=[
            pl.BlockSpec((1, c, n), lambda i: (i, 0, 0)),
            pl.BlockSpec((1, n, _K), lambda i: (i, 0, 0)),
        ],
        out_specs=pl.BlockSpec((1, c, n), lambda i: (i, 0, 0)),
        out_shape=jax.ShapeDtypeStruct((b, c, n), jnp.float32),
    )(x, idx)


def _mha_body(xq_ref, xe_ref, wq_ref, wk_ref, wv_ref, wo_ref,
              bq_ref, bk_ref, bv_ref, bo_ref, out_ref, *, dh):
    hp = pl.program_id(1)
    l = xq_ref.shape[1]
    dn = (((1,), (1,)), ((), ()))
    xq = xq_ref[0]
    xqb = xq.astype(jnp.bfloat16)
    xe = xe_ref[0]
    q2 = lax.dot_general(xqb, wq_ref[...], dn,
                         preferred_element_type=jnp.float32) + bq_ref[0]
    k2 = lax.dot_general(xe, wk_ref[...], dn,
                         preferred_element_type=jnp.float32) + bk_ref[0]
    v2 = lax.dot_general(xe, wv_ref[...], dn,
                         preferred_element_type=jnp.float32) + bv_ref[0]
    scale = 1.0 / np.sqrt(dh)
    outs = []
    for h in range(_HEADS_PER_BLOCK):
        sl = slice(h * dh, (h + 1) * dh)
        qh = (q2[:, sl] * scale).astype(jnp.bfloat16)
        kh = k2[:, sl].astype(jnp.bfloat16)
        s = lax.dot_general(qh, kh, dn, preferred_element_type=jnp.float32)
        m = jnp.max(s, axis=1, keepdims=True)
        p = jnp.exp(s - m)
        a = (p / jnp.sum(p, axis=1, keepdims=True)).astype(jnp.bfloat16)
        vh = v2[:, sl].astype(jnp.bfloat16)
        outs.append(lax.dot_general(a, vh, (((1,), (0,)), ((), ())),
                                    preferred_element_type=jnp.float32))
    o2 = jnp.concatenate(outs, axis=1).astype(jnp.bfloat16)
    proj = lax.dot_general(o2, wo_ref[...], dn,
                           preferred_element_type=jnp.float32)

    @pl.when(hp == 0)
    def _():
        out_ref[0, :l, :] = xq
        out_ref[0, l:, :] = proj + bo_ref[0][None, :]

    @pl.when(hp != 0)
    def _():
        out_ref[0, l:, :] += proj


def kernel(x, x_enc, in_proj_weight, in_proj_bias, out_proj_weight, out_proj_bias):
    b, c, n = x.shape
    s, e = x_enc.shape[1], x_enc.shape[2]
    l = c
    dh = e // _NUM_HEADS
    hb = _HEADS_PER_BLOCK
    w = hb * dh
    nhp = _NUM_HEADS // hb

    xe = x_enc.astype(jnp.bfloat16)
    wq = in_proj_weight[:e].astype(jnp.bfloat16)
    wk = in_proj_weight[e:2 * e].astype(jnp.bfloat16)
    wv = in_proj_weight[2 * e:].astype(jnp.bfloat16)
    wo = out_proj_weight.astype(jnp.bfloat16)
    bq = in_proj_bias[:e].reshape(nhp, 1, w)
    bk = in_proj_bias[e:2 * e].reshape(nhp, 1, w)
    bv = in_proj_bias[2 * e:].reshape(nhp, 1, w)
    bo = out_proj_bias.reshape(1, e)

    grid = (b, nhp)

    dist = _pairwise_dist(x)
    idx = _topk_sc(dist)
    xknn_t = _gather_mean_t(x, idx.reshape(b, n, _K))
    xq = jnp.stack([x, xknn_t], axis=3).reshape(b, c, 2 * n)

    out = pl.pallas_call(
        functools.partial(_mha_body, dh=dh),
        grid=grid,
        in_specs=[
            pl.BlockSpec((1, l, e), lambda i, j: (i, 0, 0)),
            pl.BlockSpec((1, s, e), lambda i, j: (i, 0, 0)),
            pl.BlockSpec((w, e), lambda i, j: (j, 0)),
            pl.BlockSpec((w, e), lambda i, j: (j, 0)),
            pl.BlockSpec((w, e), lambda i, j: (j, 0)),
            pl.BlockSpec((e, w), lambda i, j: (0, j)),
            pl.BlockSpec((1, 1, w), lambda i, j: (j, 0, 0)),
            pl.BlockSpec((1, 1, w), lambda i, j: (j, 0, 0)),
            pl.BlockSpec((1, 1, w), lambda i, j: (j, 0, 0)),
            pl.BlockSpec((1, e), lambda i, j: (0, 0)),
        ],
        out_specs=pl.BlockSpec((1, 2 * l, e), lambda i, j: (i, 0, 0)),
        out_shape=jax.ShapeDtypeStruct((b, 2 * l, e), jnp.float32),
        compiler_params=pltpu.CompilerParams(
            dimension_semantics=("parallel", "arbitrary"),
            vmem_limit_bytes=50 * 1024 * 1024,
        ),
    )(xq, xe, wq, wk, wv, wo, bq, bk, bv, bo)
    return out
